# baseline (speedup 1.0000x reference)
import functools

import jax
import jax.numpy as jnp
from jax import lax
from jax.experimental import pallas as pl
from jax.experimental.pallas import tpu as pltpu
from jax.experimental.pallas import tpu_sc as plsc

NC = 2
NS = 16
NW = NC * NS
L = 16


def _make_lookup(R, C, D, NBUF=2):
    bw = R // NW
    ng = bw // L
    mesh = plsc.VectorSubcoreMesh(core_axis_name="c", subcore_axis_name="s")

    @functools.partial(
        pl.kernel,
        mesh=mesh,
        out_type=jax.ShapeDtypeStruct((C, D, R), jnp.float32),
        scratch_types=[
            pltpu.VMEM((C, bw), jnp.int32),
            [pltpu.VMEM((bw,), jnp.int32) for _ in range(NBUF)],
            [pltpu.VMEM((bw, 2 * D), jnp.float32) for _ in range(NBUF)],
            [pltpu.VMEM((D, bw), jnp.float32) for _ in range(NBUF)],
            [pltpu.SemaphoreType.DMA for _ in range(NBUF)],
            [pltpu.SemaphoreType.DMA for _ in range(NBUF)],
        ],
        compiler_params=pltpu.CompilerParams(needs_layout_passes=False),
    )
    def lookup(xt_hbm, tp_hbm, out_hbm, idx_v, pidx, bufs, fms, g_sems, w_sems):
        wid = lax.axis_index("s") * NC + lax.axis_index("c")
        col0 = wid * bw
        pltpu.sync_copy(xt_hbm.at[:, pl.ds(col0, bw)], idx_v)

        iota = lax.iota(jnp.int32, L)

        def gather_start(j, b):
            for g in range(ng):
                v = idx_v[j, pl.ds(g * L, L)]
                pidx[b][pl.ds(g * L, L)] = lax.shift_right_logical(v, 1)
            pltpu.async_copy(tp_hbm.at[pidx[b]], bufs[b], g_sems[b])

        def gather_wait(b):
            pltpu.make_async_copy(tp_hbm.at[pidx[b]], bufs[b], g_sems[b]).wait()

        def write_start(j, b):
            pltpu.async_copy(
                fms[b], out_hbm.at[j, :, pl.ds(col0, bw)], w_sems[b]
            )

        def write_wait(b):
            pltpu.make_async_copy(
                fms[b], out_hbm.at[0, :, pl.ds(col0, bw)], w_sems[b]
            ).wait()

        for b in range(NBUF):
            gather_start(b, b)

        def body(jj, carry):
            for b in range(NBUF):
                j = jj + b
                gather_wait(b)

                @pl.when(j >= NBUF)
                def _():
                    write_wait(b)

                for g in range(ng):
                    v = idx_v[j, pl.ds(g * L, L)]
                    row = iota + (g * L)
                    colb = lax.bitwise_and(v, 1) * D
                    for k in range(D):
                        vals = plsc.load_gather(bufs[b], [row, colb + k])
                        fms[b][k, pl.ds(g * L, L)] = vals
                write_start(j, b)

                @pl.when(j + NBUF < C)
                def _():
                    gather_start(j + NBUF, b)

            return carry

        lax.fori_loop(0, C // NBUF, lambda t, c: body(t * NBUF, c), 0,
                      unroll=False)

        for b in range(NBUF):
            write_wait(b)

    return lookup


def kernel(x, table):
    R, C = x.shape
    V, D = table.shape
    xt = x.T.astype(jnp.int32)
    tp = table.reshape(V // 2, 2 * D)
    out_st = _make_lookup(R, C, D)(xt, tp)
    return jnp.transpose(out_st, (2, 0, 1))


# speedup vs baseline: 1.5303x; 1.5303x over previous
import functools

import jax
import jax.numpy as jnp
from jax import lax
from jax.experimental import pallas as pl
from jax.experimental.pallas import tpu as pltpu
from jax.experimental.pallas import tpu_sc as plsc

NC = 2
NS = 16
NW = NC * NS
L = 16


def _make_lookup(R, C, D, NBUF=2):
    bw = R // NW
    ng = bw // L
    mesh = plsc.VectorSubcoreMesh(core_axis_name="c", subcore_axis_name="s")

    @functools.partial(
        pl.kernel,
        mesh=mesh,
        out_type=jax.ShapeDtypeStruct((C, D, R), jnp.float32),
        scratch_types=[
            pltpu.VMEM((C, bw), jnp.int32),
            [pltpu.VMEM((bw,), jnp.int32) for _ in range(NBUF)],
            [pltpu.VMEM((bw, 2 * D), jnp.float32) for _ in range(NBUF)],
            [pltpu.VMEM((D, bw), jnp.float32) for _ in range(NBUF)],
            [pltpu.SemaphoreType.DMA for _ in range(NBUF)],
            [pltpu.SemaphoreType.DMA for _ in range(NBUF)],
        ],
        compiler_params=pltpu.CompilerParams(needs_layout_passes=False),
    )
    def lookup(xt_hbm, tp_hbm, out_hbm, idx_v, pidx, bufs, fms, g_sems, w_sems):
        wid = lax.axis_index("s") * NC + lax.axis_index("c")
        col0 = wid * bw
        pltpu.sync_copy(xt_hbm.at[:, pl.ds(col0, bw)], idx_v)

        iota = lax.iota(jnp.int32, L)

        def gather_start(j, b):
            for g in range(ng):
                v = idx_v[j, pl.ds(g * L, L)]
                pidx[b][pl.ds(g * L, L)] = lax.shift_right_logical(v, 1)
            pltpu.async_copy(tp_hbm.at[pidx[b]], bufs[b], g_sems[b])

        def gather_wait(b):
            pltpu.make_async_copy(tp_hbm.at[pidx[b]], bufs[b], g_sems[b]).wait()

        def write_start(j, b):
            pltpu.async_copy(
                fms[b], out_hbm.at[j, :, pl.ds(col0, bw)], w_sems[b]
            )

        def write_wait(b):
            pltpu.make_async_copy(
                fms[b], out_hbm.at[0, :, pl.ds(col0, bw)], w_sems[b]
            ).wait()

        for b in range(NBUF):
            gather_start(b, b)

        def body(jj, carry):
            for b in range(NBUF):
                j = jj + b
                gather_wait(b)

                @pl.when(j >= NBUF)
                def _():
                    write_wait(b)

                for g in range(ng):
                    v = idx_v[j, pl.ds(g * L, L)]
                    row = iota + (g * L)
                    colb = lax.bitwise_and(v, 1) * D

                    @plsc.parallel_loop(0, D, 1, unroll=8)
                    def _(k, _row=row, _colb=colb, _b=b, _g=g):
                        vals = plsc.load_gather(
                            bufs[_b], [_row, _colb + k]
                        )
                        fms[_b][k, pl.ds(_g * L, L)] = vals
                write_start(j, b)

                @pl.when(j + NBUF < C)
                def _():
                    gather_start(j + NBUF, b)

            return carry

        lax.fori_loop(0, C // NBUF, lambda t, c: body(t * NBUF, c), 0,
                      unroll=False)

        for b in range(NBUF):
            write_wait(b)

    return lookup


def kernel(x, table):
    R, C = x.shape
    V, D = table.shape
    xt = x.T.astype(jnp.int32)
    tp = table.reshape(V // 2, 2 * D)
    out_st = _make_lookup(R, C, D)(xt, tp)
    return jnp.transpose(out_st, (2, 0, 1))


# speedup vs baseline: 1.5306x; 1.0001x over previous
import functools

import jax
import jax.numpy as jnp
from jax import lax
from jax.experimental import pallas as pl
from jax.experimental.pallas import tpu as pltpu
from jax.experimental.pallas import tpu_sc as plsc

NC = 2
NS = 16
NW = NC * NS
L = 16


def _make_lookup(R, C, D, NBUF=2):
    bw = R // NW
    ng = bw // L
    mesh = plsc.VectorSubcoreMesh(core_axis_name="c", subcore_axis_name="s")

    @functools.partial(
        pl.kernel,
        mesh=mesh,
        out_type=jax.ShapeDtypeStruct((C, D, R), jnp.float32),
        scratch_types=[
            pltpu.VMEM((C, bw), jnp.int32),
            [pltpu.VMEM((bw,), jnp.int32) for _ in range(NBUF)],
            [pltpu.VMEM((bw, 2 * D), jnp.float32) for _ in range(NBUF)],
            [pltpu.VMEM((D, bw), jnp.float32) for _ in range(NBUF)],
            [pltpu.SemaphoreType.DMA for _ in range(NBUF)],
            [pltpu.SemaphoreType.DMA for _ in range(NBUF)],
        ],
        compiler_params=pltpu.CompilerParams(needs_layout_passes=False),
    )
    def lookup(xt_hbm, tp_hbm, out_hbm, idx_v, pidx, bufs, fms, g_sems, w_sems):
        wid = lax.axis_index("s") * NC + lax.axis_index("c")
        col0 = wid * bw
        pltpu.sync_copy(xt_hbm.at[:, pl.ds(col0, bw)], idx_v)

        iota = lax.iota(jnp.int32, L)

        def gather_start(j, b):
            for g in range(ng):
                v = idx_v[j, pl.ds(g * L, L)]
                pidx[b][pl.ds(g * L, L)] = lax.shift_right_logical(v, 1)
            pltpu.async_copy(tp_hbm.at[pidx[b]], bufs[b], g_sems[b])

        def gather_wait(b):
            pltpu.make_async_copy(tp_hbm.at[pidx[b]], bufs[b], g_sems[b]).wait()

        def write_start(j, b):
            pltpu.async_copy(
                fms[b], out_hbm.at[j, :, pl.ds(col0, bw)], w_sems[b]
            )

        def write_wait(b):
            pltpu.make_async_copy(
                fms[b], out_hbm.at[0, :, pl.ds(col0, bw)], w_sems[b]
            ).wait()

        for b in range(NBUF):
            gather_start(b, b)

        def body(jj, carry):
            for b in range(NBUF):
                j = jj + b
                gather_wait(b)

                @pl.when(j >= NBUF)
                def _():
                    write_wait(b)

                for g in range(ng):
                    v = idx_v[j, pl.ds(g * L, L)]
                    row = iota + (g * L)
                    colb = lax.bitwise_and(v, 1) * D

                    @plsc.parallel_loop(0, D, 1, unroll=16)
                    def _(k, _row=row, _colb=colb, _b=b, _g=g):
                        vals = plsc.load_gather(
                            bufs[_b], [_row, _colb + k]
                        )
                        fms[_b][k, pl.ds(_g * L, L)] = vals
                write_start(j, b)

                @pl.when(j + NBUF < C)
                def _():
                    gather_start(j + NBUF, b)

            return carry

        lax.fori_loop(0, C // NBUF, lambda t, c: body(t * NBUF, c), 0,
                      unroll=False)

        for b in range(NBUF):
            write_wait(b)

    return lookup


def kernel(x, table):
    R, C = x.shape
    V, D = table.shape
    xt = x.T.astype(jnp.int32)
    tp = table.reshape(V // 2, 2 * D)
    out_st = _make_lookup(R, C, D)(xt, tp)
    return jnp.transpose(out_st, (2, 0, 1))
